# baseline (speedup 1.0000x reference)
import jax
import jax.numpy as jnp
from jax import lax
from jax.experimental import pallas as pl
from jax.experimental.pallas import tpu as pltpu
from jax.experimental.pallas import tpu_sc as plsc

D_MODEL = 64
SCALE = 8.0
NC, NS = 2, 16
NW = NC * NS
LANES = 16

VOCAB = 1000000
BAND = 256
FULL_BANDS = VOCAB // BAND
TAIL0 = FULL_BANDS * BAND
TAIL = VOCAB - TAIL0


def _iota16():
    return lax.iota(jnp.int32, 16)


def _transpose_body(tT_hbm, tailp_hbm, scratch_hbm, tbuf, obuf, tlbuf, sem_r, sem_w):
    wid = lax.axis_index("s") * NC + lax.axis_index("c")
    per_w = FULL_BANDS // NW
    n_i = per_w + jnp.where(wid < FULL_BANDS - per_w * NW, 1, 0)

    def issue_read(i):
        t = wid + i * NW
        sl = i & 1
        for jj in range(D_MODEL // 8):
            pltpu.async_copy(
                tT_hbm.at[pl.ds(jj * 8, 8), pl.ds(t * BAND, BAND)],
                tbuf.at[pl.ds(sl * 64 + jj * 8, 8)],
                sem_r.at[sl],
            )

    def wait_read(sl):
        for jj in range(D_MODEL // 8):
            pltpu.make_async_copy(
                tT_hbm.at[pl.ds(0, 8), pl.ds(0, BAND)],
                tbuf.at[pl.ds(sl * 64 + jj * 8, 8)],
                sem_r.at[sl],
            ).wait()

    def wait_write(sl):
        pltpu.make_async_copy(
            obuf.at[pl.ds(sl * BAND, BAND)],
            scratch_hbm.at[pl.ds(0, BAND)],
            sem_w.at[sl],
        ).wait()

    issue_read(0)

    @pl.loop(0, n_i)
    def _band(i):
        sl = i & 1
        t = wid + i * NW

        @pl.when(i + 1 < n_i)
        def _():
            issue_read(i + 1)

        wait_read(sl)

        @pl.when(i >= 2)
        def _():
            wait_write(sl)

        @pl.loop(0, BAND, unroll=8)
        def _col(vv):
            col = jnp.full((16,), vv, dtype=jnp.int32)
            for jb in range(D_MODEL // LANES):
                rows = sl * 64 + jb * LANES + _iota16()
                vals = plsc.load_gather(tbuf, [rows, col])
                obuf[sl * BAND + vv, pl.ds(jb * LANES, LANES)] = vals

        pltpu.async_copy(
            obuf.at[pl.ds(sl * BAND, BAND)],
            scratch_hbm.at[pl.ds(t * BAND, BAND)],
            sem_w.at[sl],
        )

    wait_write(0)
    wait_write(1)

    @pl.when(wid == 0)
    def _tail():
        pltpu.sync_copy(tailp_hbm, tlbuf)
        pltpu.sync_copy(tlbuf, scratch_hbm.at[pl.ds(TAIL0, TAIL)])


def _gather_body(xT_hbm, scratch_hbm, out_hbm, xv, rows_v, obuf, sem_g, sem_w):
    wid = lax.axis_index("s") * NC + lax.axis_index("c")
    b0 = wid * 128
    n_cp = (xv.shape[0] * 8) // 2

    for cb in range(xv.shape[0]):
        pltpu.sync_copy(
            xT_hbm.at[pl.ds(cb * 8, 8), pl.ds(b0, 128)], xv.at[cb]
        )

    def issue_gather(cp):
        sl = cp & 1
        for h in range(2):
            c = cp * 2 + h
            pltpu.async_copy(
                scratch_hbm.at[xv.at[c >> 3, c & 7]],
                rows_v.at[pl.ds((sl * 2 + h) * 128, 128)],
                sem_g.at[sl],
            )

    def wait_gather(sl):
        for h in range(2):
            pltpu.make_async_copy(
                scratch_hbm.at[pl.ds(0, 128)],
                rows_v.at[pl.ds((sl * 2 + h) * 128, 128)],
                sem_g.at[sl],
            ).wait()

    def wait_write(sl):
        for h in range(2):
            pltpu.make_async_copy(
                obuf.at[pl.ds((sl * 2 + h) * 64, 64)],
                out_hbm.at[0, pl.ds(0, 64), pl.ds(b0, 128)],
                sem_w.at[sl],
            ).wait()

    issue_gather(0)

    @pl.loop(0, n_cp)
    def _colpair(cp):
        sl = cp & 1

        @pl.when(cp + 1 < n_cp)
        def _():
            issue_gather(cp + 1)

        wait_gather(sl)

        @pl.when(cp >= 2)
        def _():
            wait_write(sl)

        for h in range(2):
            rbase = (sl * 2 + h) * 128
            obase = (sl * 2 + h) * 64

            @pl.loop(0, 128, unroll=8)
            def _row(b):
                bcol = jnp.full((16,), b, dtype=jnp.int32)
                for jb in range(D_MODEL // LANES):
                    vals = rows_v[rbase + b, pl.ds(jb * LANES, LANES)] * SCALE
                    jrows = obase + jb * LANES + _iota16()
                    plsc.store_scatter(obuf, [jrows, bcol], vals)

        for h in range(2):
            c = cp * 2 + h
            pltpu.async_copy(
                obuf.at[pl.ds((sl * 2 + h) * 64, 64)],
                out_hbm.at[c, pl.ds(0, 64), pl.ds(b0, 128)],
                sem_w.at[sl],
            )

    wait_write(0)
    wait_write(1)


def kernel(x, table):
    n_b, n_c = x.shape
    mesh = plsc.VectorSubcoreMesh(core_axis_name="c", subcore_axis_name="s")
    params = pltpu.CompilerParams(
        use_tc_tiling_on_sc=True, needs_layout_passes=False
    )

    tT = table.T
    tailp = jnp.pad(
        table[TAIL0:, :], ((0, 0), (0, 128 - D_MODEL))
    )

    scratch = pl.kernel(
        _transpose_body,
        out_type=jax.ShapeDtypeStruct((VOCAB, 128), jnp.float32),
        mesh=mesh,
        scratch_types=[
            pltpu.VMEM((2 * D_MODEL, BAND), jnp.float32),
            pltpu.VMEM((2 * BAND, 128), jnp.float32),
            pltpu.VMEM((TAIL, 128), jnp.float32),
            pltpu.SemaphoreType.DMA((2,)),
            pltpu.SemaphoreType.DMA((2,)),
        ],
        compiler_params=params,
    )(tT, tailp)

    outT = pl.kernel(
        _gather_body,
        out_type=jax.ShapeDtypeStruct((n_c, D_MODEL, n_b), jnp.float32),
        mesh=mesh,
        scratch_types=[
            pltpu.VMEM((n_c // 8, 8, 128), jnp.int32),
            pltpu.VMEM((4 * 128, 128), jnp.float32),
            pltpu.VMEM((4 * D_MODEL, 128), jnp.float32),
            pltpu.SemaphoreType.DMA((2,)),
            pltpu.SemaphoreType.DMA((2,)),
        ],
        compiler_params=params,
    )(x.T, scratch)

    return outT.transpose(2, 0, 1)


# speedup vs baseline: 1.7342x; 1.7342x over previous
import jax
import jax.numpy as jnp
from jax import lax
from jax.experimental import pallas as pl
from jax.experimental.pallas import tpu as pltpu
from jax.experimental.pallas import tpu_sc as plsc

D_MODEL = 64
SCALE = 8.0
NC, NS = 2, 16
NW = NC * NS
LANES = 16

VOCAB = 1000000
BAND = 256
FULL_BANDS = VOCAB // BAND
TAIL0 = FULL_BANDS * BAND
TAIL = VOCAB - TAIL0


def _iota16():
    return lax.iota(jnp.int32, 16)


def _transpose_body(tT_hbm, tailp_hbm, scratch_hbm, tbuf, obuf, tlbuf, sem_r, sem_w):
    wid = lax.axis_index("s") * NC + lax.axis_index("c")
    per_w = FULL_BANDS // NW
    n_i = per_w + jnp.where(wid < FULL_BANDS - per_w * NW, 1, 0)

    def issue_read(i):
        t = wid + i * NW
        sl = i & 1
        for jj in range(D_MODEL // 8):
            pltpu.async_copy(
                tT_hbm.at[pl.ds(jj * 8, 8), pl.ds(t * BAND, BAND)],
                tbuf.at[pl.ds(sl * 64 + jj * 8, 8)],
                sem_r.at[sl],
            )

    def wait_read(sl):
        for jj in range(D_MODEL // 8):
            pltpu.make_async_copy(
                tT_hbm.at[pl.ds(0, 8), pl.ds(0, BAND)],
                tbuf.at[pl.ds(sl * 64 + jj * 8, 8)],
                sem_r.at[sl],
            ).wait()

    def wait_write(sl):
        pltpu.make_async_copy(
            obuf.at[pl.ds(sl * BAND, BAND)],
            scratch_hbm.at[pl.ds(0, BAND)],
            sem_w.at[sl],
        ).wait()

    issue_read(0)

    @pl.loop(0, n_i)
    def _band(i):
        sl = i & 1
        t = wid + i * NW

        @pl.when(i + 1 < n_i)
        def _():
            issue_read(i + 1)

        wait_read(sl)

        @pl.when(i >= 2)
        def _():
            wait_write(sl)

        @plsc.parallel_loop(0, BAND, unroll=8)
        def _col(vv):
            col = jnp.full((16,), vv, dtype=jnp.int32)
            for jb in range(D_MODEL // LANES):
                rows = sl * 64 + jb * LANES + _iota16()
                vals = plsc.load_gather(tbuf, [rows, col])
                obuf[sl * BAND + vv, pl.ds(jb * LANES, LANES)] = vals

        pltpu.async_copy(
            obuf.at[pl.ds(sl * BAND, BAND)],
            scratch_hbm.at[pl.ds(t * BAND, BAND)],
            sem_w.at[sl],
        )

    wait_write(0)
    wait_write(1)

    @pl.when(wid == 0)
    def _tail():
        pltpu.sync_copy(tailp_hbm, tlbuf)
        pltpu.sync_copy(tlbuf, scratch_hbm.at[pl.ds(TAIL0, TAIL)])


def _gather_body(xT_hbm, scratch_hbm, out_hbm, xv, rows_v, obuf, sem_g, sem_w):
    wid = lax.axis_index("s") * NC + lax.axis_index("c")
    b0 = wid * 128
    n_cp = (xv.shape[0] * 8) // 2

    for cb in range(xv.shape[0]):
        pltpu.sync_copy(
            xT_hbm.at[pl.ds(cb * 8, 8), pl.ds(b0, 128)], xv.at[cb]
        )

    def issue_gather(cp):
        sl = cp & 1
        for h in range(2):
            c = cp * 2 + h
            pltpu.async_copy(
                scratch_hbm.at[xv.at[c >> 3, c & 7]],
                rows_v.at[pl.ds((sl * 2 + h) * 128, 128)],
                sem_g.at[sl],
            )

    def wait_gather(sl):
        for h in range(2):
            pltpu.make_async_copy(
                scratch_hbm.at[pl.ds(0, 128)],
                rows_v.at[pl.ds((sl * 2 + h) * 128, 128)],
                sem_g.at[sl],
            ).wait()

    def wait_write(sl):
        for h in range(2):
            pltpu.make_async_copy(
                obuf.at[pl.ds((sl * 2 + h) * 64, 64)],
                out_hbm.at[0, pl.ds(0, 64), pl.ds(b0, 128)],
                sem_w.at[sl],
            ).wait()

    issue_gather(0)

    @pl.loop(0, n_cp)
    def _colpair(cp):
        sl = cp & 1

        @pl.when(cp + 1 < n_cp)
        def _():
            issue_gather(cp + 1)

        wait_gather(sl)

        @pl.when(cp >= 2)
        def _():
            wait_write(sl)

        for h in range(2):
            rbase = (sl * 2 + h) * 128
            obase = (sl * 2 + h) * 64

            @plsc.parallel_loop(0, 128, unroll=8)
            def _row(b):
                bcol = jnp.full((16,), b, dtype=jnp.int32)
                for jb in range(D_MODEL // LANES):
                    vals = rows_v[rbase + b, pl.ds(jb * LANES, LANES)] * SCALE
                    jrows = obase + jb * LANES + _iota16()
                    plsc.store_scatter(obuf, [jrows, bcol], vals)

        for h in range(2):
            c = cp * 2 + h
            pltpu.async_copy(
                obuf.at[pl.ds((sl * 2 + h) * 64, 64)],
                out_hbm.at[c, pl.ds(0, 64), pl.ds(b0, 128)],
                sem_w.at[sl],
            )

    wait_write(0)
    wait_write(1)


def kernel(x, table):
    n_b, n_c = x.shape
    mesh = plsc.VectorSubcoreMesh(core_axis_name="c", subcore_axis_name="s")
    params = pltpu.CompilerParams(
        use_tc_tiling_on_sc=True, needs_layout_passes=False
    )

    tT = table.T
    tailp = jnp.pad(
        table[TAIL0:, :], ((0, 0), (0, 128 - D_MODEL))
    )

    scratch = pl.kernel(
        _transpose_body,
        out_type=jax.ShapeDtypeStruct((VOCAB, 128), jnp.float32),
        mesh=mesh,
        scratch_types=[
            pltpu.VMEM((2 * D_MODEL, BAND), jnp.float32),
            pltpu.VMEM((2 * BAND, 128), jnp.float32),
            pltpu.VMEM((TAIL, 128), jnp.float32),
            pltpu.SemaphoreType.DMA((2,)),
            pltpu.SemaphoreType.DMA((2,)),
        ],
        compiler_params=params,
    )(tT, tailp)

    outT = pl.kernel(
        _gather_body,
        out_type=jax.ShapeDtypeStruct((n_c, D_MODEL, n_b), jnp.float32),
        mesh=mesh,
        scratch_types=[
            pltpu.VMEM((n_c // 8, 8, 128), jnp.int32),
            pltpu.VMEM((4 * 128, 128), jnp.float32),
            pltpu.VMEM((4 * D_MODEL, 128), jnp.float32),
            pltpu.SemaphoreType.DMA((2,)),
            pltpu.SemaphoreType.DMA((2,)),
        ],
        compiler_params=params,
    )(x.T, scratch)

    return outT.transpose(2, 0, 1)
